# baseline (speedup 1.0000x reference)
import functools

import jax
import jax.numpy as jnp
from jax import lax
from jax.experimental import pallas as pl
from jax.experimental.pallas import tpu as pltpu
from jax.experimental.pallas import tpu_sc as plsc

BATCH = 16384
D = 64
L = 16
IDX_CHUNK = 128


def _build_sc_call():
    mesh = plsc.VectorSubcoreMesh(core_axis_name="c", subcore_axis_name="s")
    nc, ns = mesh.num_cores, mesh.num_subcores
    nw = nc * ns
    b_per_w = BATCH // nw
    n_chunks = b_per_w // IDX_CHUNK
    n_groups = b_per_w // L

    @functools.partial(
        pl.kernel,
        out_type=jax.ShapeDtypeStruct((BATCH,), jnp.float32),
        mesh=mesh,
        scratch_types=[
            pltpu.VMEM((b_per_w,), jnp.int32),
            pltpu.VMEM((b_per_w,), jnp.int32),
            pltpu.VMEM((b_per_w, D), jnp.float32),
            pltpu.VMEM((b_per_w, D), jnp.float32),
            pltpu.VMEM((b_per_w,), jnp.float32),
            pltpu.VMEM((D,), jnp.float32),
            pltpu.VMEM((L,), jnp.float32),
            pltpu.SemaphoreType.DMA,
        ],
        compiler_params=pltpu.CompilerParams(use_tc_tiling_on_sc=False),
    )
    def neumf_kernel(uidx_hbm, iidx_hbm, uemb_hbm, iemb_hbm, w_hbm, b_hbm,
                     out_hbm, idx_u, idx_i, u_rows, i_rows, out_v, w_v, b_v,
                     sem):
        wid = lax.axis_index("s") * nc + lax.axis_index("c")
        base = wid * b_per_w
        pltpu.sync_copy(uidx_hbm.at[pl.ds(base, b_per_w)], idx_u)
        pltpu.sync_copy(iidx_hbm.at[pl.ds(base, b_per_w)], idx_i)
        pltpu.sync_copy(w_hbm, w_v)
        pltpu.sync_copy(b_hbm, b_v)

        copies = []
        for c in range(n_chunks):
            sl = pl.ds(c * IDX_CHUNK, IDX_CHUNK)
            copies.append(
                pltpu.async_copy(uemb_hbm.at[idx_u.at[sl]], u_rows.at[sl], sem))
            copies.append(
                pltpu.async_copy(iemb_hbm.at[idx_i.at[sl]], i_rows.at[sl], sem))
        for cp in copies:
            cp.wait()

        w_chunks = [w_v[pl.ds(c * L, L)] for c in range(D // L)]
        bd = b_v[...]
        lane = lax.iota(jnp.int32, L)
        perms = [jnp.bitwise_xor(lane, d) for d in (1, 2, 4, 8)]

        dnums = lax.GatherDimensionNumbers(
            offset_dims=(), collapsed_slice_dims=(0,), start_index_map=(0,))

        def permute(s, p):
            return lax.gather(
                s, p[:, None], dnums, (1,),
                mode=lax.GatherScatterMode.PROMISE_IN_BOUNDS)

        def lane_sum(s):
            for p in perms:
                s = s + permute(s, p)
            return s

        def group_body(g, carry):
            acc = jnp.zeros((L,), jnp.float32)
            for j in range(L):
                r = g * L + j
                s = bd
                for c in range(D // L):
                    cs = pl.ds(c * L, L)
                    s = s + u_rows[r, cs] * i_rows[r, cs] * w_chunks[c]
                acc = jnp.where(lane == j, lane_sum(s), acc)
            out_v[pl.ds(g * L, L)] = acc
            return carry

        lax.fori_loop(0, n_groups, group_body, 0)
        pltpu.sync_copy(out_v, out_hbm.at[pl.ds(base, b_per_w)])

    return neumf_kernel


def kernel(users_index, items_index, user_mf_emb, item_mf_emb, W_pred, b_pred):
    w_flat = W_pred.reshape(D).astype(jnp.float32)
    b_lane = jnp.full((L,), b_pred[0] / L, dtype=jnp.float32)
    call = _build_sc_call()
    out = call(users_index.astype(jnp.int32), items_index.astype(jnp.int32),
               user_mf_emb, item_mf_emb, w_flat, b_lane)
    return out.reshape(BATCH, 1)
